# baseline (speedup 1.0000x reference)
import functools

import jax
import jax.numpy as jnp
from jax import lax
from jax.experimental import pallas as pl
from jax.experimental.pallas import tpu as pltpu
from jax.experimental.pallas import tpu_sc as plsc

_DIM = 128
_NUM_WORKERS = 32
_CHUNK = 128


def _normalize_table_body(emb_ref, out_ref):
    e = emb_ref[...]
    ss = jnp.sum(e * e, axis=1, keepdims=True)
    out_ref[...] = e * lax.rsqrt(ss)


def _normalize_table(embedding):
    return pl.pallas_call(
        _normalize_table_body,
        out_shape=jax.ShapeDtypeStruct(embedding.shape, jnp.float32),
    )(embedding)


@functools.lru_cache(maxsize=None)
def _make_gather(n_pad, chunks_per_worker):
    b_per_w = n_pad // _NUM_WORKERS
    mesh = plsc.VectorSubcoreMesh(core_axis_name="c", subcore_axis_name="s")

    @functools.partial(
        pl.kernel,
        mesh=mesh,
        out_type=jax.ShapeDtypeStruct((n_pad, _DIM), jnp.float32),
        scratch_types=[
            pltpu.VMEM((_CHUNK,), jnp.int32),
            pltpu.VMEM((_CHUNK, _DIM), jnp.float32),
            pltpu.SemaphoreType.DMA,
        ],
    )
    def gather(table_hbm, idx_hbm, out_hbm, idx_v, rows_v, sem):
        wid = lax.axis_index("s") * 2 + lax.axis_index("c")
        base = wid * b_per_w

        def body(k, carry):
            off = base + k * _CHUNK
            pltpu.sync_copy(idx_hbm.at[pl.ds(off, _CHUNK)], idx_v)
            pltpu.async_copy(table_hbm.at[idx_v], rows_v, sem).wait()
            pltpu.sync_copy(rows_v, out_hbm.at[pl.ds(off, _CHUNK)])
            return carry

        lax.fori_loop(0, chunks_per_worker, body, 0)

    return gather


def kernel(x, embedding):
    n = x.shape[0]
    table = _normalize_table(embedding.astype(jnp.float32))
    cpw = -(-n // (_NUM_WORKERS * _CHUNK))
    n_pad = _NUM_WORKERS * _CHUNK * cpw
    xi = x.astype(jnp.int32)
    if n_pad != n:
        xi = jnp.concatenate([xi, jnp.zeros((n_pad - n,), jnp.int32)])
    out = _make_gather(n_pad, cpw)(table, xi)
    return out[:n]


# speedup vs baseline: 1.0773x; 1.0773x over previous
import functools

import jax
import jax.numpy as jnp
from jax import lax
from jax.experimental import pallas as pl
from jax.experimental.pallas import tpu as pltpu
from jax.experimental.pallas import tpu_sc as plsc

_DIM = 128
_NUM_WORKERS = 32
_CHUNK = 128
_NBUF = 5


def _normalize_table_body(emb_ref, out_ref):
    e = emb_ref[...]
    ss = jnp.sum(e * e, axis=1, keepdims=True)
    out_ref[...] = e * lax.rsqrt(ss)


def _normalize_table(embedding):
    return pl.pallas_call(
        _normalize_table_body,
        out_shape=jax.ShapeDtypeStruct(embedding.shape, jnp.float32),
    )(embedding)


@functools.lru_cache(maxsize=None)
def _make_gather(n_pad, cpw):
    ngroups = cpw // _NBUF
    mesh = plsc.VectorSubcoreMesh(core_axis_name="c", subcore_axis_name="s")

    @functools.partial(
        pl.kernel,
        mesh=mesh,
        out_type=jax.ShapeDtypeStruct((n_pad, _DIM), jnp.float32),
        scratch_types=[
            pltpu.VMEM((cpw, _CHUNK), jnp.int32),
            pltpu.VMEM((_NBUF, _CHUNK, _DIM), jnp.float32),
        ]
        + [pltpu.SemaphoreType.DMA] * (2 * _NBUF),
    )
    def gather(table_hbm, idx_hbm, out_hbm, idx_v, rows_v, *sems):
        semg = sems[:_NBUF]
        semw = sems[_NBUF:]
        wid = lax.axis_index("s") * 2 + lax.axis_index("c")
        row0 = wid * cpw
        pltpu.sync_copy(idx_hbm.at[wid], idx_v)

        def start_gather(k, b):
            pltpu.async_copy(table_hbm.at[idx_v.at[k]], rows_v.at[b], semg[b])

        def wait_gather(b):
            pltpu.make_async_copy(
                out_hbm.at[pl.ds(0, _CHUNK)], rows_v.at[b], semg[b]
            ).wait()

        def wait_writeback(b):
            pltpu.make_async_copy(
                rows_v.at[b], out_hbm.at[pl.ds(0, _CHUNK)], semw[b]
            ).wait()

        for b in range(_NBUF):
            start_gather(b, b)

        def body(g, carry):
            for b in range(_NBUF):
                k = g * _NBUF + b
                out_off = (row0 + k) * _CHUNK
                wait_gather(b)
                pltpu.async_copy(
                    rows_v.at[b], out_hbm.at[pl.ds(out_off, _CHUNK)], semw[b]
                )

                @pl.when(g < ngroups - 1)
                def _():
                    wait_writeback(b)
                    start_gather(k + _NBUF, b)

            return carry

        lax.fori_loop(0, ngroups, body, 0)
        for b in range(_NBUF):
            wait_writeback(b)

    return gather


def kernel(x, embedding):
    n = x.shape[0]
    table = _normalize_table(embedding.astype(jnp.float32))
    grain = _NUM_WORKERS * _CHUNK * _NBUF
    n_pad = grain * (-(-n // grain))
    cpw = n_pad // (_NUM_WORKERS * _CHUNK)
    xi = x.astype(jnp.int32)
    if n_pad != n:
        xi = jnp.concatenate([xi, jnp.zeros((n_pad - n,), jnp.int32)])
    out = _make_gather(n_pad, cpw)(
        table, xi.reshape(_NUM_WORKERS, cpw, _CHUNK)
    )
    return out[:n]


# speedup vs baseline: 1.5667x; 1.4543x over previous
import functools

import jax
import jax.numpy as jnp
from jax import lax
from jax.experimental import pallas as pl
from jax.experimental.pallas import tpu as pltpu
from jax.experimental.pallas import tpu_sc as plsc

_DIM = 128
_TABLE_ROWS = 120
_NUM_WORKERS = 32
_CHUNK = 128
_NBUF = 5


def _normalize_table_body(emb_ref, out_ref):
    e = emb_ref[...]
    ss = jnp.sum(e * e, axis=1, keepdims=True)
    out_ref[...] = e * lax.rsqrt(ss)


def _normalize_table(embedding):
    return pl.pallas_call(
        _normalize_table_body,
        out_shape=jax.ShapeDtypeStruct(embedding.shape, jnp.float32),
    )(embedding)


@functools.lru_cache(maxsize=None)
def _make_gather(n_pad, cpw):
    ngroups = cpw // _NBUF
    mesh = plsc.VectorSubcoreMesh(core_axis_name="c", subcore_axis_name="s")

    @functools.partial(
        pl.kernel,
        mesh=mesh,
        out_type=jax.ShapeDtypeStruct((n_pad, _DIM), jnp.float32),
        scratch_types=[
            pltpu.VMEM((cpw, _CHUNK), jnp.int32),
            pltpu.VMEM((_NBUF, _CHUNK, _DIM), jnp.float32),
        ]
        + [pltpu.SemaphoreType.DMA] * (2 * _NBUF),
    )
    def gather(table_hbm, idx_hbm, out_hbm, idx_v, rows_v, *sems):
        semg = sems[:_NBUF]
        semw = sems[_NBUF:]
        wid = lax.axis_index("s") * 2 + lax.axis_index("c")
        row0 = wid * cpw
        pltpu.sync_copy(idx_hbm.at[wid], idx_v)

        off = jnp.full((16,), _TABLE_ROWS, jnp.int32) * wid

        def off_body(t, c):
            j = t // (_CHUNK // 16)
            cc = t % (_CHUNK // 16)
            idx_v[j, pl.ds(cc * 16, 16)] = idx_v[j, pl.ds(cc * 16, 16)] + off
            return c

        lax.fori_loop(0, cpw * (_CHUNK // 16), off_body, 0)

        def start_gather(k, b):
            pltpu.async_copy(table_hbm.at[idx_v.at[k]], rows_v.at[b], semg[b])

        def wait_gather(b):
            pltpu.make_async_copy(
                out_hbm.at[pl.ds(0, _CHUNK)], rows_v.at[b], semg[b]
            ).wait()

        def wait_writeback(b):
            pltpu.make_async_copy(
                rows_v.at[b], out_hbm.at[pl.ds(0, _CHUNK)], semw[b]
            ).wait()

        for b in range(_NBUF):
            start_gather(b, b)

        def body(g, carry):
            for b in range(_NBUF):
                k = g * _NBUF + b
                out_off = (row0 + k) * _CHUNK
                wait_gather(b)
                pltpu.async_copy(
                    rows_v.at[b], out_hbm.at[pl.ds(out_off, _CHUNK)], semw[b]
                )

                @pl.when(g < ngroups - 1)
                def _():
                    wait_writeback(b)
                    start_gather(k + _NBUF, b)

            return carry

        lax.fori_loop(0, ngroups, body, 0)
        for b in range(_NBUF):
            wait_writeback(b)

    return gather


def kernel(x, embedding):
    n = x.shape[0]
    table = _normalize_table(embedding.astype(jnp.float32))
    grain = _NUM_WORKERS * _CHUNK * _NBUF
    n_pad = grain * (-(-n // grain))
    cpw = n_pad // (_NUM_WORKERS * _CHUNK)
    xi = x.astype(jnp.int32)
    if n_pad != n:
        xi = jnp.concatenate([xi, jnp.zeros((n_pad - n,), jnp.int32)])
    table_rep = jnp.tile(table, (_NUM_WORKERS, 1))
    out = _make_gather(n_pad, cpw)(
        table_rep, xi.reshape(_NUM_WORKERS, cpw, _CHUNK)
    )
    return out[:n]


# speedup vs baseline: 2.3992x; 1.5314x over previous
import functools

import jax
import jax.numpy as jnp
from jax import lax
from jax.experimental import pallas as pl
from jax.experimental.pallas import tpu as pltpu
from jax.experimental.pallas import tpu_sc as plsc

_DIM = 128
_TABLE_ROWS = 120
_NUM_WORKERS = 32
_CHUNK = 128
_NBUF = 5
_REP_PER_W = 4


def _normalize_table_body(emb_ref, out_ref):
    e = emb_ref[...]
    ss = jnp.sum(e * e, axis=1, keepdims=True)
    out_ref[...] = e * lax.rsqrt(ss)


def _normalize_table(embedding):
    return pl.pallas_call(
        _normalize_table_body,
        out_shape=jax.ShapeDtypeStruct(embedding.shape, jnp.float32),
    )(embedding)


@functools.lru_cache(maxsize=None)
def _make_gather(n_pad, cpw):
    ngroups = cpw // _NBUF
    mesh = plsc.VectorSubcoreMesh(core_axis_name="c", subcore_axis_name="s")

    @functools.partial(
        pl.kernel,
        mesh=mesh,
        out_type=jax.ShapeDtypeStruct((n_pad, _DIM), jnp.float32),
        scratch_types=[
            pltpu.VMEM((cpw, _CHUNK), jnp.int32),
            pltpu.VMEM((_NBUF, _CHUNK, _DIM), jnp.float32),
        ]
        + [pltpu.SemaphoreType.DMA] * (2 * _NBUF),
    )
    def gather(table_hbm, idx_hbm, out_hbm, idx_v, rows_v, *sems):
        semg = sems[:_NBUF]
        semw = sems[_NBUF:]
        wid = lax.axis_index("s") * 2 + lax.axis_index("c")
        row0 = wid * cpw
        pltpu.sync_copy(idx_hbm.at[wid], idx_v)

        def off_body(t, c):
            j = t // (_CHUNK // 16)
            cc = t % (_CHUNK // 16)
            rep = wid * _REP_PER_W + j % _REP_PER_W
            off = jnp.full((16,), _TABLE_ROWS, jnp.int32) * rep
            idx_v[j, pl.ds(cc * 16, 16)] = idx_v[j, pl.ds(cc * 16, 16)] + off
            return c

        lax.fori_loop(0, cpw * (_CHUNK // 16), off_body, 0)

        def start_gather(k, b):
            pltpu.async_copy(table_hbm.at[idx_v.at[k]], rows_v.at[b], semg[b])

        def wait_gather(b):
            pltpu.make_async_copy(
                out_hbm.at[pl.ds(0, _CHUNK)], rows_v.at[b], semg[b]
            ).wait()

        def wait_writeback(b):
            pltpu.make_async_copy(
                rows_v.at[b], out_hbm.at[pl.ds(0, _CHUNK)], semw[b]
            ).wait()

        for b in range(_NBUF):
            start_gather(b, b)

        def body(g, carry):
            for b in range(_NBUF):
                k = g * _NBUF + b
                out_off = (row0 + k) * _CHUNK
                wait_gather(b)
                pltpu.async_copy(
                    rows_v.at[b], out_hbm.at[pl.ds(out_off, _CHUNK)], semw[b]
                )

                @pl.when(g < ngroups - 1)
                def _():
                    wait_writeback(b)
                    start_gather(k + _NBUF, b)

            return carry

        lax.fori_loop(0, ngroups, body, 0)
        for b in range(_NBUF):
            wait_writeback(b)

    return gather


def kernel(x, embedding):
    n = x.shape[0]
    table = _normalize_table(embedding.astype(jnp.float32))
    grain = _NUM_WORKERS * _CHUNK * _NBUF
    n_pad = grain * (-(-n // grain))
    cpw = n_pad // (_NUM_WORKERS * _CHUNK)
    xi = x.astype(jnp.int32)
    if n_pad != n:
        xi = jnp.concatenate([xi, jnp.zeros((n_pad - n,), jnp.int32)])
    table_rep = jnp.tile(table, (_NUM_WORKERS * _REP_PER_W, 1))
    out = _make_gather(n_pad, cpw)(
        table_rep, xi.reshape(_NUM_WORKERS, cpw, _CHUNK)
    )
    return out[:n]


# speedup vs baseline: 2.7530x; 1.1475x over previous
import functools

import jax
import jax.numpy as jnp
from jax import lax
from jax.experimental import pallas as pl
from jax.experimental.pallas import tpu as pltpu
from jax.experimental.pallas import tpu_sc as plsc

_DIM = 128
_TABLE_ROWS = 120
_NUM_WORKERS = 32
_CHUNK = 128
_NBUF = 5
_REP_PER_W = 8


def _normalize_table_body(emb_ref, out_ref):
    e = emb_ref[...]
    ss = jnp.sum(e * e, axis=1, keepdims=True)
    out_ref[...] = e * lax.rsqrt(ss)


def _normalize_table(embedding):
    return pl.pallas_call(
        _normalize_table_body,
        out_shape=jax.ShapeDtypeStruct(embedding.shape, jnp.float32),
    )(embedding)


@functools.lru_cache(maxsize=None)
def _make_gather(n_pad, cpw):
    ngroups = cpw // _NBUF
    mesh = plsc.VectorSubcoreMesh(core_axis_name="c", subcore_axis_name="s")

    @functools.partial(
        pl.kernel,
        mesh=mesh,
        out_type=jax.ShapeDtypeStruct((n_pad, _DIM), jnp.float32),
        scratch_types=[
            pltpu.VMEM((cpw, _CHUNK), jnp.int32),
            pltpu.VMEM((_NBUF, _CHUNK, _DIM), jnp.float32),
        ]
        + [pltpu.SemaphoreType.DMA] * (2 * _NBUF),
    )
    def gather(table_hbm, idx_hbm, out_hbm, idx_v, rows_v, *sems):
        semg = sems[:_NBUF]
        semw = sems[_NBUF:]
        wid = lax.axis_index("s") * 2 + lax.axis_index("c")
        row0 = wid * cpw
        pltpu.sync_copy(idx_hbm.at[wid], idx_v)

        def off_body(t, c):
            j = t // (_CHUNK // 16)
            cc = t % (_CHUNK // 16)
            rep = wid * _REP_PER_W + (j + cc) % _REP_PER_W
            off = jnp.full((16,), _TABLE_ROWS, jnp.int32) * rep
            idx_v[j, pl.ds(cc * 16, 16)] = idx_v[j, pl.ds(cc * 16, 16)] + off
            return c

        lax.fori_loop(0, cpw * (_CHUNK // 16), off_body, 0)

        def start_gather(k, b):
            pltpu.async_copy(table_hbm.at[idx_v.at[k]], rows_v.at[b], semg[b])

        def wait_gather(b):
            pltpu.make_async_copy(
                out_hbm.at[pl.ds(0, _CHUNK)], rows_v.at[b], semg[b]
            ).wait()

        def wait_writeback(b):
            pltpu.make_async_copy(
                rows_v.at[b], out_hbm.at[pl.ds(0, _CHUNK)], semw[b]
            ).wait()

        for b in range(_NBUF):
            start_gather(b, b)

        def body(g, carry):
            for b in range(_NBUF):
                k = g * _NBUF + b
                out_off = (row0 + k) * _CHUNK
                wait_gather(b)
                pltpu.async_copy(
                    rows_v.at[b], out_hbm.at[pl.ds(out_off, _CHUNK)], semw[b]
                )

                @pl.when(g < ngroups - 1)
                def _():
                    wait_writeback(b)
                    start_gather(k + _NBUF, b)

            return carry

        lax.fori_loop(0, ngroups, body, 0)
        for b in range(_NBUF):
            wait_writeback(b)

    return gather


def kernel(x, embedding):
    n = x.shape[0]
    table = _normalize_table(embedding.astype(jnp.float32))
    grain = _NUM_WORKERS * _CHUNK * _NBUF
    n_pad = grain * (-(-n // grain))
    cpw = n_pad // (_NUM_WORKERS * _CHUNK)
    xi = x.astype(jnp.int32)
    if n_pad != n:
        xi = jnp.concatenate([xi, jnp.zeros((n_pad - n,), jnp.int32)])
    table_rep = jnp.tile(table, (_NUM_WORKERS * _REP_PER_W, 1))
    out = _make_gather(n_pad, cpw)(
        table_rep, xi.reshape(_NUM_WORKERS, cpw, _CHUNK)
    )
    return out[:n]


# speedup vs baseline: 5.2844x; 1.9195x over previous
import functools

import jax
import jax.numpy as jnp
from jax import lax
from jax.experimental import pallas as pl
from jax.experimental.pallas import tpu as pltpu
from jax.experimental.pallas import tpu_sc as plsc

_DIM = 128
_TABLE_ROWS = 120
_NUM_WORKERS = 32
_CHUNK = 128
_NBUF = 5
_REP_PER_W = 8


def _normalize_table_body(emb_ref, out_ref):
    e = emb_ref[...]
    ss = jnp.sum(e * e, axis=1, keepdims=True)
    out_ref[...] = e * lax.rsqrt(ss)


def _normalize_table(embedding):
    return pl.pallas_call(
        _normalize_table_body,
        out_shape=jax.ShapeDtypeStruct(embedding.shape, jnp.float32),
    )(embedding)


@functools.lru_cache(maxsize=None)
def _make_gather(n):
    full = n // _CHUNK
    rem = n % _CHUNK
    base_cnt = full // _NUM_WORKERS
    extra = full % _NUM_WORKERS
    max_cnt = base_cnt + (1 if extra else 0)
    ngroups = -(-max_cnt // _NBUF)
    assert base_cnt >= _NBUF and rem % 8 == 0 and rem <= _CHUNK
    mesh = plsc.VectorSubcoreMesh(core_axis_name="c", subcore_axis_name="s")

    @functools.partial(
        pl.kernel,
        mesh=mesh,
        out_type=jax.ShapeDtypeStruct((n, _DIM), jnp.float32),
        scratch_types=[
            pltpu.VMEM((max_cnt, _CHUNK), jnp.int32),
            pltpu.VMEM((_NBUF, _CHUNK, _DIM), jnp.float32),
            pltpu.VMEM((max(rem, 8), ), jnp.int32),
            pltpu.VMEM((max(rem, 8), _DIM), jnp.float32),
        ]
        + [pltpu.SemaphoreType.DMA] * (2 * _NBUF + 2),
    )
    def gather(table_hbm, idx_hbm, out_hbm, idx_v, rows_v, idx_t, rows_t, *sems):
        semg = sems[:_NBUF]
        semw = sems[_NBUF:2 * _NBUF]
        sem_i = sems[2 * _NBUF]
        sem_t = sems[2 * _NBUF + 1]
        wid = lax.axis_index("s") * 2 + lax.axis_index("c")
        cnt = jnp.where(wid < extra, base_cnt + 1, base_cnt)

        for j in range(max_cnt):

            @pl.when(j < cnt)
            def _(j=j):
                pltpu.async_copy(
                    idx_hbm.at[pl.ds((wid + j * _NUM_WORKERS) * _CHUNK, _CHUNK)],
                    idx_v.at[j],
                    sem_i,
                )

        if rem:
            @pl.when(wid == extra)
            def _():
                pltpu.async_copy(idx_hbm.at[pl.ds(n - rem, rem)], idx_t, sem_t)

        for j in range(max_cnt):

            @pl.when(j < cnt)
            def _(j=j):
                pltpu.make_async_copy(
                    idx_hbm.at[pl.ds(0, _CHUNK)], idx_v.at[j], sem_i
                ).wait()

        def off_body(t, c):
            j = t // (_CHUNK // 16)
            cc = t % (_CHUNK // 16)
            rep = wid * _REP_PER_W + (j + cc) % _REP_PER_W
            off = jnp.full((16,), _TABLE_ROWS, jnp.int32) * rep
            idx_v[j, pl.ds(cc * 16, 16)] = idx_v[j, pl.ds(cc * 16, 16)] + off
            return c

        lax.fori_loop(0, max_cnt * (_CHUNK // 16), off_body, 0)

        if rem:
            @pl.when(wid == extra)
            def _():
                pltpu.make_async_copy(
                    idx_hbm.at[pl.ds(0, rem)], idx_t, sem_t
                ).wait()
                off_t = jnp.full((16,), _TABLE_ROWS * extra * _REP_PER_W, jnp.int32)
                for c in range(rem // 16):
                    idx_t[pl.ds(c * 16, 16)] = idx_t[pl.ds(c * 16, 16)] + off_t
                pltpu.async_copy(table_hbm.at[idx_t], rows_t, sem_t)

        def start_gather(k, b):
            pltpu.async_copy(table_hbm.at[idx_v.at[k]], rows_v.at[b], semg[b])

        def wait_gather(b):
            pltpu.make_async_copy(
                out_hbm.at[pl.ds(0, _CHUNK)], rows_v.at[b], semg[b]
            ).wait()

        def wait_writeback(b):
            pltpu.make_async_copy(
                rows_v.at[b], out_hbm.at[pl.ds(0, _CHUNK)], semw[b]
            ).wait()

        for b in range(_NBUF):
            start_gather(b, b)

        def body(g, carry):
            for b in range(_NBUF):
                k = g * _NBUF + b

                @pl.when(k < cnt)
                def _(k=k, b=b):
                    out_off = (wid + k * _NUM_WORKERS) * _CHUNK
                    wait_gather(b)
                    pltpu.async_copy(
                        rows_v.at[b], out_hbm.at[pl.ds(out_off, _CHUNK)], semw[b]
                    )

                @pl.when(k < cnt - _NBUF)
                def _(k=k, b=b):
                    wait_writeback(b)
                    start_gather(k + _NBUF, b)

            return carry

        lax.fori_loop(0, ngroups, body, 0)
        for b in range(_NBUF):
            wait_writeback(b)

        if rem:
            @pl.when(wid == extra)
            def _():
                pltpu.make_async_copy(
                    out_hbm.at[pl.ds(0, rem)], rows_t, sem_t
                ).wait()
                pltpu.sync_copy(rows_t, out_hbm.at[pl.ds(n - rem, rem)])

    return gather


def kernel(x, embedding):
    n = x.shape[0]
    table = _normalize_table(embedding.astype(jnp.float32))
    table_rep = jnp.tile(table, (_NUM_WORKERS * _REP_PER_W, 1))
    return _make_gather(n)(table_rep, x.astype(jnp.int32))


# speedup vs baseline: 7.9525x; 1.5049x over previous
import functools

import jax
import jax.numpy as jnp
from jax import lax
from jax.experimental import pallas as pl
from jax.experimental.pallas import tpu as pltpu
from jax.experimental.pallas import tpu_sc as plsc

_DIM = 128
_TABLE_ROWS = 120
_NUM_WORKERS = 32
_CHUNK = 128
_NBUF = 5
_REP_PER_W = 8


def _normalize_table_body(emb_ref, out_ref):
    e = emb_ref[...]
    ss = jnp.sum(e * e, axis=1, keepdims=True)
    out_ref[...] = e * lax.rsqrt(ss)


def _normalize_table(embedding):
    return pl.pallas_call(
        _normalize_table_body,
        out_shape=jax.ShapeDtypeStruct(embedding.shape, jnp.float32),
    )(embedding)


@functools.lru_cache(maxsize=None)
def _make_gather(n):
    full = n // _CHUNK
    rem = n % _CHUNK
    base_cnt = full // _NUM_WORKERS
    extra = full % _NUM_WORKERS
    max_cnt = base_cnt + (1 if extra else 0)
    ngroups = -(-max_cnt // _NBUF)
    assert base_cnt >= _NBUF and rem % 8 == 0 and rem <= _CHUNK
    mesh = plsc.VectorSubcoreMesh(core_axis_name="c", subcore_axis_name="s")

    @functools.partial(
        pl.kernel,
        mesh=mesh,
        out_type=jax.ShapeDtypeStruct((n, _DIM), jnp.float32),
        scratch_types=[
            pltpu.VMEM((max_cnt, _CHUNK), jnp.int32),
            pltpu.VMEM((_NBUF, _CHUNK, _DIM), jnp.float32),
            pltpu.VMEM((max(rem, 8), ), jnp.int32),
            pltpu.VMEM((max(rem, 8), _DIM), jnp.float32),
            pltpu.VMEM_SHARED((_TABLE_ROWS, _DIM), jnp.float32),
        ]
        + [pltpu.SemaphoreType.DMA] * (2 * _NBUF + 2),
    )
    def gather(table_hbm, idx_hbm, out_hbm, idx_v, rows_v, idx_t, rows_t,
               table_sh, *sems):
        semg = sems[:_NBUF]
        semw = sems[_NBUF:2 * _NBUF]
        sem_i = sems[2 * _NBUF]
        sem_t = sems[2 * _NBUF + 1]
        wid = lax.axis_index("s") * 2 + lax.axis_index("c")
        cnt = jnp.where(wid < extra, base_cnt + 1, base_cnt)

        @pl.when(lax.axis_index("s") == 0)
        def _():
            pltpu.sync_copy(table_hbm, table_sh)

        for j in range(max_cnt):

            @pl.when(j < cnt)
            def _(j=j):
                pltpu.async_copy(
                    idx_hbm.at[pl.ds((wid + j * _NUM_WORKERS) * _CHUNK, _CHUNK)],
                    idx_v.at[j],
                    sem_i,
                )

        if rem:
            @pl.when(wid == extra)
            def _():
                pltpu.async_copy(idx_hbm.at[pl.ds(n - rem, rem)], idx_t, sem_t)

        for j in range(max_cnt):

            @pl.when(j < cnt)
            def _(j=j):
                pltpu.make_async_copy(
                    idx_hbm.at[pl.ds(0, _CHUNK)], idx_v.at[j], sem_i
                ).wait()

        plsc.subcore_barrier()

        if rem:
            @pl.when(wid == extra)
            def _():
                pltpu.make_async_copy(
                    idx_hbm.at[pl.ds(0, rem)], idx_t, sem_t
                ).wait()
                pltpu.async_copy(table_sh.at[idx_t], rows_t, sem_t)

        def start_gather(k, b):
            pltpu.async_copy(table_sh.at[idx_v.at[k]], rows_v.at[b], semg[b])

        def wait_gather(b):
            pltpu.make_async_copy(
                out_hbm.at[pl.ds(0, _CHUNK)], rows_v.at[b], semg[b]
            ).wait()

        def wait_writeback(b):
            pltpu.make_async_copy(
                rows_v.at[b], out_hbm.at[pl.ds(0, _CHUNK)], semw[b]
            ).wait()

        for b in range(_NBUF):
            start_gather(b, b)

        def body(g, carry):
            for b in range(_NBUF):
                k = g * _NBUF + b

                @pl.when(k < cnt)
                def _(k=k, b=b):
                    out_off = (wid + k * _NUM_WORKERS) * _CHUNK
                    wait_gather(b)
                    pltpu.async_copy(
                        rows_v.at[b], out_hbm.at[pl.ds(out_off, _CHUNK)], semw[b]
                    )

                @pl.when(k < cnt - _NBUF)
                def _(k=k, b=b):
                    wait_writeback(b)
                    start_gather(k + _NBUF, b)

            return carry

        lax.fori_loop(0, ngroups, body, 0)
        for b in range(_NBUF):
            wait_writeback(b)

        if rem:
            @pl.when(wid == extra)
            def _():
                pltpu.make_async_copy(
                    out_hbm.at[pl.ds(0, rem)], rows_t, sem_t
                ).wait()
                pltpu.sync_copy(rows_t, out_hbm.at[pl.ds(n - rem, rem)])

    return gather


def kernel(x, embedding):
    n = x.shape[0]
    table = _normalize_table(embedding.astype(jnp.float32))
    return _make_gather(n)(table, x.astype(jnp.int32))
